# baseline (speedup 1.0000x reference)
import functools

import jax
import jax.numpy as jnp
from jax import lax
from jax.experimental import pallas as pl
from jax.experimental.pallas import tpu as pltpu
from jax.experimental.pallas import tpu_sc as plsc

_NEW_TOKEN_ID = 1000000
_VOCAB = 1000000
_D = 64
_DP = 128

_NC = 2
_NS = 16
_NW = _NC * _NS

_EPC = 2


_TBLK = 2048


def _prep_table(table_t):
    v = table_t.shape[1]

    def body(x_ref, o_ref):
        o_ref[:, : _D] = lax.dot_general(
            x_ref[...],
            jnp.eye(_D, dtype=jnp.float32),
            (((0,), (0,)), ((), ())),
            preferred_element_type=jnp.float32,
        )

    return pl.pallas_call(
        body,
        grid=((v + _TBLK - 1) // _TBLK,),
        in_specs=[pl.BlockSpec((_D, _TBLK), lambda i: (0, i))],
        out_specs=pl.BlockSpec((_TBLK, _DP), lambda i: (i, 0)),
        out_shape=jax.ShapeDtypeStruct((v, _DP), jnp.float32),
    )(table_t)


@functools.partial(jax.jit, static_argnums=(3, 4))
def _lookup(ids, table_p, new_row, batch, seq):
    e_per_w = batch // _NW
    chunk = _EPC * seq
    n_chunks = e_per_w // _EPC
    n_pairs = n_chunks // 2
    n_groups = chunk // 16
    mesh = plsc.VectorSubcoreMesh(core_axis_name="c", subcore_axis_name="s")

    @functools.partial(
        pl.kernel,
        mesh=mesh,
        out_type=jax.ShapeDtypeStruct((batch, seq, _DP), jnp.float32),
        scratch_types=[
            pltpu.VMEM((chunk,), jnp.int32),
            pltpu.VMEM((chunk,), jnp.int32),
            pltpu.VMEM((chunk,), jnp.int32),
            pltpu.VMEM((chunk,), jnp.int32),
            pltpu.VMEM((chunk, _DP), jnp.float32),
            pltpu.VMEM((chunk, _DP), jnp.float32),
            pltpu.VMEM((_D,), jnp.float32),
            pltpu.SMEM((2,), jnp.int32),
            pltpu.SemaphoreType.DMA,
            pltpu.SemaphoreType.DMA,
            pltpu.SemaphoreType.DMA,
            pltpu.SemaphoreType.DMA,
            pltpu.SemaphoreType.DMA,
            pltpu.SemaphoreType.DMA,
        ],
        compiler_params=pltpu.CompilerParams(
            needs_layout_passes=False, use_tc_tiling_on_sc=True
        ),
    )
    def k(ids_hbm, table_hbm, new_hbm, out_hbm, ids0, ids1, msk0, msk1,
          rows0, rows1, new_v, flags, isem0, isem1, gsem0, gsem1,
          osem0, osem1):
        wid = lax.axis_index("s") * _NC + lax.axis_index("c")
        webase = wid * e_per_w
        idb = (ids0, ids1)
        mskb = (msk0, msk1)
        rows = (rows0, rows1)
        isem = (isem0, isem1)
        gsem = (gsem0, gsem1)
        osem = (osem0, osem1)

        pltpu.sync_copy(new_hbm, new_v)

        def ids_cp(ci, b):
            base = (webase + ci * _EPC) * seq
            return pltpu.make_async_copy(
                ids_hbm.at[pl.ds(base, chunk)], idb[b], isem[b]
            )

        def pass1(ci, b):
            mx = None
            for g in range(n_groups):
                idv = idb[b][pl.ds(g * 16, 16)]
                pltpu.async_copy(
                    table_hbm.at[jnp.minimum(idv, _VOCAB - 1)],
                    rows[b].at[pl.ds(g * 16, 16)],
                    gsem[b],
                )
                mskb[b][pl.ds(g * 16, 16)] = jnp.where(
                    idv == _NEW_TOKEN_ID, 1, 0
                )
                mx = idv if mx is None else jnp.maximum(mx, idv)
            flags[b] = jnp.max(mx)

        def drain_gathers(b):
            pltpu.make_async_copy(
                table_hbm.at[pl.ds(0, chunk)], rows[b], gsem[b]
            ).wait()

        def out_cps(ci, b):
            e = webase + ci * _EPC
            return [
                pltpu.make_async_copy(
                    rows[b].at[pl.ds(j * seq, seq)], out_hbm.at[e + j], osem[b]
                )
                for j in range(_EPC)
            ]

        def fixup(ci, b):
            @pl.when(flags[b] >= _NEW_TOKEN_ID)
            def _fix():
                liota = lax.iota(jnp.int32, 16)
                for g in range(n_groups):
                    m = mskb[b][pl.ds(g * 16, 16)]
                    @pl.when(jnp.max(m) > 0)
                    def _grp(g=g, m=m):
                        def w_body(mr):
                            lane = jnp.min(jnp.where(mr > 0, liota, 16))
                            row = g * 16 + lane
                            for q in range(_D // 16):
                                rows[b][row, pl.ds(q * 16, 16)] = (
                                    new_v[pl.ds(q * 16, 16)]
                                )
                            return jnp.where(liota == lane, 0, mr)
                        lax.while_loop(
                            lambda mr: jnp.max(mr) > 0, w_body, m
                        )

        for b in range(2):
            ids_cp(b, b).start()
        for b in range(2):
            ids_cp(b, b).wait()
            pass1(b, b)
        for b in range(2):
            @pl.when(2 + b < n_chunks)
            def _pre(b=b):
                ids_cp(2 + b, b).start()

        def pair(p, carry):
            for b in range(2):
                ci = 2 * p + b
                nci = ci + 2
                drain_gathers(b)
                fixup(ci, b)
                cps = out_cps(ci, b)
                for cp in cps:
                    cp.start()
                for cp in cps:
                    cp.wait()
                @pl.when(nci < n_chunks)
                def _prep():
                    ids_cp(nci, b).wait()
                    pass1(nci, b)
                @pl.when(nci + 2 < n_chunks)
                def _pref():
                    ids_cp(nci + 2, b).start()
            return carry

        lax.fori_loop(0, n_pairs, pair, 0)

    return k(ids, table_p, new_row)


def kernel(input_ids, base_weight, new_weight):
    b, s = input_ids.shape
    ids = input_ids.reshape(b * s).astype(jnp.int32)
    table_p = _prep_table(base_weight.T)
    out = _lookup(ids, table_p, new_weight.reshape(_D), b, s)
    return out[..., :_D]


# speedup vs baseline: 1.0230x; 1.0230x over previous
import functools

import jax
import jax.numpy as jnp
from jax import lax
from jax.experimental import pallas as pl
from jax.experimental.pallas import tpu as pltpu
from jax.experimental.pallas import tpu_sc as plsc

_NEW_TOKEN_ID = 1000000
_VOCAB = 1000000
_D = 64
_DP = 128

_NC = 2
_NS = 16
_NW = _NC * _NS

_EPC = 2


_TBLK = 2048


def _prep_table(table_t):
    v = table_t.shape[1]

    def body(x_ref, o_ref):
        o_ref[:, : _D] = x_ref[...].T

    return pl.pallas_call(
        body,
        grid=((v + _TBLK - 1) // _TBLK,),
        in_specs=[pl.BlockSpec((_D, _TBLK), lambda i: (0, i))],
        out_specs=pl.BlockSpec((_TBLK, _DP), lambda i: (i, 0)),
        out_shape=jax.ShapeDtypeStruct((v, _DP), jnp.float32),
    )(table_t)


@functools.partial(jax.jit, static_argnums=(3, 4))
def _lookup(ids, table_p, new_row, batch, seq):
    e_per_w = batch // _NW
    chunk = _EPC * seq
    n_chunks = e_per_w // _EPC
    n_pairs = n_chunks // 2
    n_groups = chunk // 16
    mesh = plsc.VectorSubcoreMesh(core_axis_name="c", subcore_axis_name="s")

    @functools.partial(
        pl.kernel,
        mesh=mesh,
        out_type=jax.ShapeDtypeStruct((batch, seq, _DP), jnp.float32),
        scratch_types=[
            pltpu.VMEM((chunk,), jnp.int32),
            pltpu.VMEM((chunk,), jnp.int32),
            pltpu.VMEM((chunk,), jnp.int32),
            pltpu.VMEM((chunk,), jnp.int32),
            pltpu.VMEM((chunk, _DP), jnp.float32),
            pltpu.VMEM((chunk, _DP), jnp.float32),
            pltpu.VMEM((_D,), jnp.float32),
            pltpu.SMEM((2,), jnp.int32),
            pltpu.SemaphoreType.DMA,
            pltpu.SemaphoreType.DMA,
            pltpu.SemaphoreType.DMA,
            pltpu.SemaphoreType.DMA,
            pltpu.SemaphoreType.DMA,
            pltpu.SemaphoreType.DMA,
        ],
        compiler_params=pltpu.CompilerParams(
            needs_layout_passes=False, use_tc_tiling_on_sc=True
        ),
    )
    def k(ids_hbm, table_hbm, new_hbm, out_hbm, ids0, ids1, msk0, msk1,
          rows0, rows1, new_v, flags, isem0, isem1, gsem0, gsem1,
          osem0, osem1):
        wid = lax.axis_index("s") * _NC + lax.axis_index("c")
        webase = wid * e_per_w
        idb = (ids0, ids1)
        mskb = (msk0, msk1)
        rows = (rows0, rows1)
        isem = (isem0, isem1)
        gsem = (gsem0, gsem1)
        osem = (osem0, osem1)

        pltpu.sync_copy(new_hbm, new_v)

        def ids_cp(ci, b):
            base = (webase + ci * _EPC) * seq
            return pltpu.make_async_copy(
                ids_hbm.at[pl.ds(base, chunk)], idb[b], isem[b]
            )

        def pass1(ci, b):
            mx = None
            for g in range(n_groups):
                idv = idb[b][pl.ds(g * 16, 16)]
                pltpu.async_copy(
                    table_hbm.at[jnp.minimum(idv, _VOCAB - 1)],
                    rows[b].at[pl.ds(g * 16, 16)],
                    gsem[b],
                )
                mskb[b][pl.ds(g * 16, 16)] = jnp.where(
                    idv == _NEW_TOKEN_ID, 1, 0
                )
                mx = idv if mx is None else jnp.maximum(mx, idv)
            flags[b] = jnp.max(mx)

        def drain_gathers(b):
            pltpu.make_async_copy(
                table_hbm.at[pl.ds(0, chunk)], rows[b], gsem[b]
            ).wait()

        def out_cps(ci, b):
            e = webase + ci * _EPC
            return [
                pltpu.make_async_copy(
                    rows[b].at[pl.ds(j * seq, seq)], out_hbm.at[e + j], osem[b]
                )
                for j in range(_EPC)
            ]

        def fixup(ci, b):
            @pl.when(flags[b] >= _NEW_TOKEN_ID)
            def _fix():
                liota = lax.iota(jnp.int32, 16)
                for g in range(n_groups):
                    m = mskb[b][pl.ds(g * 16, 16)]
                    @pl.when(jnp.max(m) > 0)
                    def _grp(g=g, m=m):
                        def w_body(mr):
                            lane = jnp.min(jnp.where(mr > 0, liota, 16))
                            row = g * 16 + lane
                            for q in range(_D // 16):
                                rows[b][row, pl.ds(q * 16, 16)] = (
                                    new_v[pl.ds(q * 16, 16)]
                                )
                            return jnp.where(liota == lane, 0, mr)
                        lax.while_loop(
                            lambda mr: jnp.max(mr) > 0, w_body, m
                        )

        for b in range(2):
            ids_cp(b, b).start()
        for b in range(2):
            ids_cp(b, b).wait()
            pass1(b, b)
        for b in range(2):
            @pl.when(2 + b < n_chunks)
            def _pre(b=b):
                ids_cp(2 + b, b).start()

        def pair(p, carry):
            for b in range(2):
                ci = 2 * p + b
                nci = ci + 2
                drain_gathers(b)
                fixup(ci, b)
                cps = out_cps(ci, b)
                for cp in cps:
                    cp.start()
                for cp in cps:
                    cp.wait()
                @pl.when(nci < n_chunks)
                def _prep():
                    ids_cp(nci, b).wait()
                    pass1(nci, b)
                @pl.when(nci + 2 < n_chunks)
                def _pref():
                    ids_cp(nci + 2, b).start()
            return carry

        lax.fori_loop(0, n_pairs, pair, 0)

    return k(ids, table_p, new_row)


def kernel(input_ids, base_weight, new_weight):
    b, s = input_ids.shape
    ids = input_ids.reshape(b * s).astype(jnp.int32)
    table_p = _prep_table(base_weight.T)
    out = _lookup(ids, table_p, new_weight.reshape(_D), b, s)
    return out[..., :_D]
